# baseline (speedup 1.0000x reference)
import functools

import jax
import jax.numpy as jnp
import numpy as np
from jax import lax
from jax.experimental import pallas as pl
from jax.experimental.pallas import tpu as pltpu
from jax.experimental.pallas import tpu_sc as plsc

N_NODES = 50000
N_EDGES = 800000
HID = 64
FILT = 64
NG = 50
CUTOFF = 4.5
NUM_INTERACTIONS = 3
SHIFT = float(np.log(2.0))

NC = 2
NS = 16
LANES = 16

HALF = N_NODES // NC
AGG_ROWS = 12544
SLICE = AGG_ROWS // NS
TRASH = AGG_ROWS - 4

CHUNK = 40
CHUNKS_PER_WK = N_EDGES // CHUNK // (NC * NS)

EC = 80
ECHUNKS = N_NODES // EC
NW = NC * NS
EC_BASE = ECHUNKS // NW
EC_REM = ECHUNKS % NW


def _sc_mesh():
    return plsc.VectorSubcoreMesh(core_axis_name="c", subcore_axis_name="s")


def _filters_body(ea_ref, offs_ref, wnn_ref, bnn_ref, w0_ref, w1_ref, w2_ref):
    ea = ea_ref[...]
    offs = offs_ref[...]
    dd = CUTOFF / (NG - 1)
    coeff = -0.5 / (dd * dd)
    g = jnp.exp(coeff * (ea - offs) ** 2)
    r = jnp.dot(g, wnn_ref[...], preferred_element_type=jnp.float32)
    r = r + bnn_ref[...]
    cw = 0.5 * (jnp.cos(ea * (np.pi / CUTOFF)) + 1.0)
    r = r * cw
    w0_ref[...] = r[:, :FILT]
    w1_ref[...] = r[:, FILT:2 * FILT]
    w2_ref[...] = r[:, 2 * FILT:]


def _edge_filters(edge_attr, wnn_all, bnn_all, offs):
    BE = 4000
    grid = (N_EDGES // BE,)
    out = jax.ShapeDtypeStruct((N_EDGES, FILT), jnp.float32)
    return pl.pallas_call(
        _filters_body,
        grid=grid,
        in_specs=[
            pl.BlockSpec((BE, 1), lambda i: (i, 0)),
            pl.BlockSpec((1, NG), lambda i: (0, 0)),
            pl.BlockSpec((NG, 3 * FILT), lambda i: (0, 0)),
            pl.BlockSpec((1, 3 * FILT), lambda i: (0, 0)),
        ],
        out_specs=[pl.BlockSpec((BE, FILT), lambda i: (i, 0))] * 3,
        out_shape=[out, out, out],
    )(edge_attr, offs, wnn_all, bnn_all)


def _embed_body(ids_hbm, embed_hbm, x_hbm, idx_v, rows_v, sem):
    c = lax.axis_index("c")
    s = lax.axis_index("s")
    w = s * NC + c
    n = EC_BASE + (w < EC_REM).astype(jnp.int32)

    def body(i, carry):
        cid = w + i * NW
        base = cid * EC
        pltpu.sync_copy(ids_hbm.at[pl.ds(base, EC)], idx_v)
        pltpu.async_copy(embed_hbm.at[idx_v], rows_v, sem).wait()
        pltpu.sync_copy(rows_v, x_hbm.at[pl.ds(base, EC)])
        return carry

    lax.fori_loop(0, n, body, 0)


def _embed_gather(ids, embed_pad):
    kfn = functools.partial(
        pl.kernel,
        out_type=jax.ShapeDtypeStruct((N_NODES, 2 * HID), jnp.float32),
        mesh=_sc_mesh(),
        scratch_types=[
            pltpu.VMEM((EC,), jnp.int32),
            pltpu.VMEM((EC, 2 * HID), jnp.float32),
            pltpu.SemaphoreType.DMA,
        ],
    )(_embed_body)
    return kfn(ids, embed_pad)


def _edge_body(h_hbm, w_hbm, src_hbm, out_hbm, src_v, hrows, wrows, me, gsem):
    c = lax.axis_index("c")
    s = lax.axis_index("s")
    w = s * NC + c

    def chunk(i, carry):
        base = (w * CHUNKS_PER_WK + i) * CHUNK
        pltpu.sync_copy(src_hbm.at[pl.ds(base, CHUNK)], src_v)
        gather = pltpu.async_copy(h_hbm.at[src_v], hrows, gsem)
        pltpu.sync_copy(w_hbm.at[pl.ds(base, CHUNK)], wrows)
        gather.wait()

        def mul(r, carry2):
            for k4 in range(HID // LANES):
                sl = pl.ds(k4 * LANES, LANES)
                me[r, sl] = hrows[r, sl] * wrows[r, sl]
            return carry2

        lax.fori_loop(0, CHUNK, mul, 0)
        pltpu.sync_copy(me, out_hbm.at[pl.ds(base, CHUNK)])
        return carry

    lax.fori_loop(0, CHUNKS_PER_WK, chunk, 0)


def _edge_msgs(h, w_l, src):
    kfn = functools.partial(
        pl.kernel,
        out_type=jax.ShapeDtypeStruct((N_EDGES, HID), jnp.float32),
        mesh=_sc_mesh(),
        scratch_types=[
            pltpu.VMEM((CHUNK,), jnp.int32),
            pltpu.VMEM((CHUNK, 2 * HID), jnp.float32),
            pltpu.VMEM((CHUNK, HID), jnp.float32),
            pltpu.VMEM((CHUNK, HID), jnp.float32),
            pltpu.SemaphoreType.DMA,
        ],
    )(_edge_body)
    return kfn(h, w_l, src)


def _seg_body(dst_ref, msg_ref, acc_ref):
    @pl.when(pl.program_id(0) == 0)
    def _():
        acc_ref[...] = jnp.zeros_like(acc_ref)

    def body(e, carry):
        d = dst_ref[0, 0, e]
        acc_ref[pl.ds(d, 1), :] = acc_ref[pl.ds(d, 1), :] \
            + msg_ref[pl.ds(e, 1), :]
        return carry

    lax.fori_loop(0, BSEG, body, 0)


BSEG = 2000


def _segment_sum(msg, dst_row):
    return pl.pallas_call(
        _seg_body,
        grid=(N_EDGES // BSEG,),
        in_specs=[
            pl.BlockSpec((1, 1, BSEG), lambda i: (i, 0, 0),
                         memory_space=pltpu.SMEM),
            pl.BlockSpec((BSEG, HID), lambda i: (i, 0)),
        ],
        out_specs=pl.BlockSpec((N_NODES, HID), lambda i: (0, 0)),
        out_shape=jax.ShapeDtypeStruct((N_NODES, HID), jnp.float32),
    )(dst_row, msg)


def _h_body(x_ref, win_ref, bin_ref, h_ref):
    xv = x_ref[...][:, :HID]
    hv = jnp.dot(xv, win_ref[...],
                 preferred_element_type=jnp.float32) + bin_ref[...]
    h_ref[...] = jnp.concatenate([hv, jnp.zeros_like(hv)], axis=1)


def _node_matmul(x_pad, win, bin_row):
    BN = 2000
    return pl.pallas_call(
        _h_body,
        grid=(N_NODES // BN,),
        in_specs=[
            pl.BlockSpec((BN, 2 * HID), lambda i: (i, 0)),
            pl.BlockSpec((HID, FILT), lambda i: (0, 0)),
            pl.BlockSpec((1, FILT), lambda i: (0, 0)),
        ],
        out_specs=pl.BlockSpec((BN, 2 * FILT), lambda i: (i, 0)),
        out_shape=jax.ShapeDtypeStruct((N_NODES, 2 * FILT), jnp.float32),
    )(x_pad, win, bin_row)


def _softplus(z):
    return jnp.maximum(z, 0.0) + jnp.log1p(jnp.exp(-jnp.abs(z)))


def _tail_fused_body(agg_ref, x_ref, wout_ref, bout_ref, wlin_ref, blin_ref,
                     win_ref, bin_ref, xo_ref, ho_ref):
    z = jnp.dot(agg_ref[...], wout_ref[...],
                preferred_element_type=jnp.float32) + bout_ref[...]
    sp = _softplus(z) - SHIFT
    xn = x_ref[...][:, :HID] + jnp.dot(
        sp, wlin_ref[...], preferred_element_type=jnp.float32) + blin_ref[...]
    xo_ref[...] = jnp.concatenate([xn, jnp.zeros_like(xn)], axis=1)
    hn = jnp.dot(xn, win_ref[...],
                 preferred_element_type=jnp.float32) + bin_ref[...]
    ho_ref[...] = jnp.concatenate([hn, jnp.zeros_like(hn)], axis=1)


def _tail_final_body(agg_ref, x_ref, wout_ref, bout_ref, wlin_ref, blin_ref,
                     xo_ref):
    z = jnp.dot(agg_ref[...], wout_ref[...],
                preferred_element_type=jnp.float32) + bout_ref[...]
    sp = _softplus(z) - SHIFT
    xo_ref[...] = x_ref[...][:, :HID] + jnp.dot(
        sp, wlin_ref[...], preferred_element_type=jnp.float32) + blin_ref[...]


def _tail_fused(agg, x_pad, wout, bout, wlin, blin, win_next, bin_next):
    BN = 2000
    mat = pl.BlockSpec((HID, HID), lambda i: (0, 0))
    row = pl.BlockSpec((1, HID), lambda i: (0, 0))
    big = pl.BlockSpec((BN, HID), lambda i: (i, 0))
    wide = pl.BlockSpec((BN, 2 * HID), lambda i: (i, 0))
    out = jax.ShapeDtypeStruct((N_NODES, 2 * HID), jnp.float32)
    return pl.pallas_call(
        _tail_fused_body,
        grid=(N_NODES // BN,),
        in_specs=[big, wide, mat, row, mat, row, mat, row],
        out_specs=[wide, wide],
        out_shape=[out, out],
    )(agg, x_pad, wout, bout, wlin, blin, win_next, bin_next)


def _tail_final(agg, x_pad, wout, bout, wlin, blin):
    BN = 2000
    mat = pl.BlockSpec((HID, HID), lambda i: (0, 0))
    row = pl.BlockSpec((1, HID), lambda i: (0, 0))
    big = pl.BlockSpec((BN, HID), lambda i: (i, 0))
    wide = pl.BlockSpec((BN, 2 * HID), lambda i: (i, 0))
    return pl.pallas_call(
        _tail_final_body,
        grid=(N_NODES // BN,),
        in_specs=[big, wide, mat, row, mat, row],
        out_specs=big,
        out_shape=jax.ShapeDtypeStruct((N_NODES, HID), jnp.float32),
    )(agg, x_pad, wout, bout, wlin, blin)


def kernel(atomic_numbers, edge_index, edge_attr, embed, Win, bin_, Wnn, bnn,
           Wout, bout, Wlin, blin):
    ids = atomic_numbers.astype(jnp.int32)
    src = edge_index[0].astype(jnp.int32)
    dst = edge_index[1].astype(jnp.int32)

    offs = jnp.linspace(0.0, CUTOFF, NG, dtype=jnp.float32).reshape(1, NG)
    wnn_all = jnp.transpose(Wnn, (1, 0, 2)).reshape(NG, 3 * FILT)
    bnn_all = bnn.reshape(1, 3 * FILT)

    w0, w1, w2 = _edge_filters(edge_attr, wnn_all, bnn_all, offs)
    ws = (w0, w1, w2)

    embed_pad = jnp.concatenate(
        [embed, jnp.zeros_like(embed)], axis=1)
    x = _embed_gather(ids, embed_pad)
    h = _node_matmul(x, Win[0], bin_[0].reshape(1, HID))

    dst_row = dst.reshape(N_EDGES // BSEG, 1, BSEG)
    for l in range(NUM_INTERACTIONS):
        msg = _edge_msgs(h, ws[l], src)
        agg = _segment_sum(msg, dst_row)
        if l + 1 < NUM_INTERACTIONS:
            x, h = _tail_fused(agg, x, Wout[l], bout[l].reshape(1, HID),
                               Wlin[l], blin[l].reshape(1, HID),
                               Win[l + 1], bin_[l + 1].reshape(1, HID))
        else:
            x = _tail_final(agg, x, Wout[l], bout[l].reshape(1, HID),
                            Wlin[l], blin[l].reshape(1, HID))
    return x
